# baseline (speedup 1.0000x reference)
import jax
import jax.numpy as jnp
from jax.experimental import pallas as pl


def kernel(bank, idx, val):
    B = idx.shape[0]
    perm = jnp.argsort(idx, stable=True)
    sidx = idx[perm]
    last = jnp.concatenate([sidx[1:] != sidx[:-1], jnp.array([True])])
    tgt = jnp.where(last, sidx, bank.shape[0])
    return bank.at[tgt].set(val[perm], mode="drop")


# speedup vs baseline: 1.1797x; 1.1797x over previous
import jax
import jax.numpy as jnp
from jax import lax
from jax.experimental import pallas as pl
from jax.experimental.pallas import tpu as pltpu
from jax.experimental.pallas import tpu_sc as plsc
from jax._src.pallas import mpmd as _mpmd

N = 100000
D = 64
B = 4096
NC = 2
NS = 16
NW = NC * NS
RPW = N // NW
NGV = B // 16
K = 16


def _shift_dup_mask(v, lanes):
    later = jnp.zeros((16,), jnp.bool_)
    for k in range(1, 16):
        ids = jnp.minimum(lanes + k, 15)
        sh = jnp.take_along_axis(v, ids, axis=0)
        later = later | ((v == sh) & ((lanes + k) < 16))
    return later


def _scatter_body(copy_hbm, idx_hbm, val_hbm, out_hbm,
                  idx_v, sel_idx, sel_pos, fin_idx, fin_pos,
                  rows_v, stamp, sem_g, sem_s):
    del copy_hbm
    w = lax.axis_index("s") * NC + lax.axis_index("c")
    base = w * RPW
    lim = base + RPW
    lanes = lax.iota(jnp.int32, 16)

    pltpu.sync_copy(idx_hbm, idx_v)

    def scan_body(k, cnt):
        v = idx_v[pl.ds(k * 16, 16)]
        m = (v >= base) & (v < lim)
        pos = k * 16 + lanes
        ones = jnp.where(m, 1, 0)
        dst = cnt + plsc.cumsum(ones) - 1
        dst = jnp.where(m, dst, 0)
        plsc.store_scatter(sel_idx, [dst], v, mask=m)
        plsc.store_scatter(sel_pos, [dst], pos, mask=m)
        return cnt + jnp.sum(ones)

    cnt = lax.fori_loop(0, NGV, scan_body, jnp.int32(0), unroll=4)
    nsel = (cnt + 15) // 16

    def stamp_body(g, _):
        off = g * 16
        v = sel_idx[pl.ds(off, 16)]
        p = sel_pos[pl.ds(off, 16)]
        valid = (off + lanes) < cnt
        vc = jnp.where(valid, v, -1)
        keep = valid & ~_shift_dup_mask(vc, lanes)
        sel_idx[pl.ds(off, 16)] = jnp.where(keep, vc, -1)
        r = jnp.where(keep, vc - base, 0)
        plsc.store_scatter(stamp, [r], p, mask=keep)
        return 0

    lax.fori_loop(0, nsel, stamp_body, 0)

    def winner_body(g, wcnt):
        off = g * 16
        v = sel_idx[pl.ds(off, 16)]
        p = sel_pos[pl.ds(off, 16)]
        ok = v >= 0
        r = jnp.where(ok, v - base, 0)
        s = plsc.load_gather(stamp, [r], mask=ok)
        win = ok & (s == p)
        ones = jnp.where(win, 1, 0)
        dst = wcnt + plsc.cumsum(ones) - 1
        dst = jnp.where(win, dst, 0)
        plsc.store_scatter(fin_idx, [dst], v, mask=win)
        plsc.store_scatter(fin_pos, [dst], p, mask=win)
        return wcnt + jnp.sum(ones)

    wcnt = lax.fori_loop(0, nsel, winner_body, jnp.int32(0))

    def group_body(g, _):
        i0 = g * K
        ng = jnp.minimum(wcnt - i0, K)

        def fire_gather(l, _):
            p = fin_pos[pl.ds(i0 + l, 16)][0]
            pltpu.async_copy(val_hbm.at[pl.ds(p, 1)], rows_v.at[pl.ds(l, 1)],
                             sem_g)
            return 0

        lax.fori_loop(0, ng, fire_gather, 0)

        def wait_gather(l, _):
            pltpu.make_async_copy(val_hbm.at[pl.ds(0, 1)],
                                  rows_v.at[pl.ds(0, 1)], sem_g).wait()
            return 0

        lax.fori_loop(0, ng, wait_gather, 0)

        def fire_scatter(l, _):
            t = fin_idx[pl.ds(i0 + l, 16)][0]
            pltpu.async_copy(rows_v.at[pl.ds(l, 1)], out_hbm.at[pl.ds(t, 1)],
                             sem_s)
            return 0

        lax.fori_loop(0, ng, fire_scatter, 0)

        def wait_scatter(l, _):
            pltpu.make_async_copy(rows_v.at[pl.ds(0, 1)],
                                  out_hbm.at[pl.ds(0, 1)], sem_s).wait()
            return 0

        lax.fori_loop(0, ng, wait_scatter, 0)
        return 0

    lax.fori_loop(0, (wcnt + K - 1) // K, group_body, 0)


def kernel(bank, idx, val):
    mesh = plsc.VectorSubcoreMesh(core_axis_name="c", subcore_axis_name="s")
    scatter = _mpmd._mpmd_map(
        [(mesh, _scatter_body)],
        out_types=jax.ShapeDtypeStruct((N, D), jnp.float32),
        input_output_aliases={0: 0},
        scratch_types=[
            pltpu.VMEM((B,), jnp.int32),
            pltpu.VMEM((B + 16,), jnp.int32),
            pltpu.VMEM((B + 16,), jnp.int32),
            pltpu.VMEM((B + 16,), jnp.int32),
            pltpu.VMEM((B + 16,), jnp.int32),
            pltpu.VMEM((K, D), jnp.float32),
            pltpu.VMEM((RPW,), jnp.int32),
            pltpu.SemaphoreType.DMA,
            pltpu.SemaphoreType.DMA,
        ],
        compiler_params=pltpu.CompilerParams(needs_layout_passes=False),
        name="sc_bank_scatter",
    )
    fresh = bank + 0.0
    return scatter(fresh, idx, val)


# speedup vs baseline: 1.6727x; 1.4179x over previous
import jax
import jax.numpy as jnp
from jax import lax
from jax.experimental import pallas as pl
from jax.experimental.pallas import tpu as pltpu
from jax.experimental.pallas import tpu_sc as plsc
from jax._src.pallas import mpmd as _mpmd

N = 100000
D = 64
B = 4096
NC = 2
NS = 16
NW = NC * NS
NGV = B // 16
CW = 256
NCHK = (N + CW - 1) // CW
KMAX = (NCHK + NW - 1) // NW
TAIL_A = 128
TAIL_B = N - (NCHK - 1) * CW - TAIL_A
PPS = B // NS
SLOTS = KMAX * CW


def _shift_dup_mask(v, lanes):
    later = jnp.zeros((16,), jnp.bool_)
    for k in range(1, 16):
        ids = jnp.minimum(lanes + k, 15)
        sh = jnp.take_along_axis(v, ids, axis=0)
        later = later | ((v == sh) & ((lanes + k) < 16))
    return later


def _body(bankT_hbm, idx_hbm, valT_hbm, outT_hbm,
          pend_sh, idx_v, sel_idx, sel_pos, stamp,
          vblk, flatbuf, wcol, wpos, rowbuf,
          cb0, cb1, tba, tbb, si0, si1, so0, so1):
    s = lax.axis_index("s")
    w = s * NC + lax.axis_index("c")
    lanes = lax.iota(jnp.int32, 16)

    for h in range(2):
        p0 = s * PPS + h * 128
        pltpu.sync_copy(valT_hbm.at[:, pl.ds(p0, 128)], vblk)

        def xpose(c, _):
            csp = jnp.full((16,), c, jnp.int32)
            for q in range(4):
                x = plsc.load_gather(vblk, [q * 16 + lanes, csp])
                flatbuf[pl.ds(c * D + q * 16, 16)] = x
            return 0

        lax.fori_loop(0, 128, xpose, 0)
        pltpu.sync_copy(flatbuf, pend_sh.at[pl.ds(p0 * D, 128 * D)])

    pltpu.sync_copy(idx_hbm, idx_v)

    def initstamp(t, _):
        stamp[pl.ds(t * 16, 16)] = jnp.full((16,), -1, jnp.int32)
        return 0

    lax.fori_loop(0, SLOTS // 16, initstamp, 0)

    def scan_body(k, cnt):
        v = idx_v[pl.ds(k * 16, 16)]
        m = ((v >> 8) & 31) == w
        pos = k * 16 + lanes
        ones = jnp.where(m, 1, 0)
        dst = cnt + plsc.cumsum(ones) - 1
        dst = jnp.where(m, dst, 0)
        plsc.store_scatter(sel_idx, [dst], v, mask=m)
        plsc.store_scatter(sel_pos, [dst], pos, mask=m)
        return cnt + jnp.sum(ones)

    cnt = lax.fori_loop(0, NGV, scan_body, jnp.int32(0), unroll=4)
    nsel = (cnt + 15) // 16

    def stamp_body(g, _):
        off = g * 16
        v = sel_idx[pl.ds(off, 16)]
        p = sel_pos[pl.ds(off, 16)]
        valid = (off + lanes) < cnt
        vc = jnp.where(valid, v, -1)
        keep = valid & ~_shift_dup_mask(vc, lanes)
        slot = (vc >> 13) * CW + (vc & (CW - 1))
        slot = jnp.where(keep, slot, 0)
        plsc.store_scatter(stamp, [slot], p, mask=keep)
        return 0

    lax.fori_loop(0, nsel, stamp_body, 0)
    plsc.subcore_barrier()

    bufs = [cb0, cb1]
    sins = [si0, si1]
    souts = [so0, so1]

    def fire_in(c, b, tail):
        if tail:
            pltpu.async_copy(bankT_hbm.at[:, pl.ds((NCHK - 1) * CW, TAIL_A)],
                             tba, sins[b])
            pltpu.async_copy(
                bankT_hbm.at[:, pl.ds((NCHK - 1) * CW + TAIL_A, TAIL_B)],
                tbb, sins[b])
        else:
            pltpu.async_copy(bankT_hbm.at[:, pl.ds(c * CW, CW)], bufs[b],
                             sins[b])

    def wait_in(b, tail):
        if tail:
            pltpu.make_async_copy(bankT_hbm.at[:, pl.ds(0, TAIL_A)],
                                  tba, sins[b]).wait()
            pltpu.make_async_copy(
                bankT_hbm.at[:, pl.ds((NCHK - 1) * CW + TAIL_A, TAIL_B)],
                tbb, sins[b]).wait()
        else:
            pltpu.make_async_copy(bankT_hbm.at[:, pl.ds(0, CW)], bufs[b],
                                  sins[b]).wait()

    def fire_out(c, b, tail):
        if tail:
            pltpu.async_copy(tba,
                             outT_hbm.at[:, pl.ds((NCHK - 1) * CW, TAIL_A)],
                             souts[b])
            pltpu.async_copy(
                tbb,
                outT_hbm.at[:, pl.ds((NCHK - 1) * CW + TAIL_A, TAIL_B)],
                souts[b])
        else:
            pltpu.async_copy(bufs[b], outT_hbm.at[:, pl.ds(c * CW, CW)],
                             souts[b])

    def wait_out(b, tail):
        if tail:
            pltpu.make_async_copy(tba, outT_hbm.at[:, pl.ds(0, TAIL_A)],
                                  souts[b]).wait()
            pltpu.make_async_copy(
                tbb,
                outT_hbm.at[:, pl.ds((NCHK - 1) * CW + TAIL_A, TAIL_B)],
                souts[b]).wait()
        else:
            pltpu.make_async_copy(bufs[b], outT_hbm.at[:, pl.ds(0, CW)],
                                  souts[b]).wait()

    def patch_chunk(k, b, tail=False):
        def walk(t, cw_):
            sl = stamp[pl.ds(k * CW + t * 16, 16)]
            m = sl >= 0
            ones = jnp.where(m, 1, 0)
            dst = cw_ + plsc.cumsum(ones) - 1
            dst = jnp.where(m, dst, 0)
            plsc.store_scatter(wcol, [dst], t * 16 + lanes, mask=m)
            plsc.store_scatter(wpos, [dst], sl, mask=m)
            return cw_ + jnp.sum(ones)

        cw_ = lax.fori_loop(0, CW // 16, walk, jnp.int32(0))

        def apply_one(i, _):
            col = wcol[pl.ds(i, 16)][0]
            p = wpos[pl.ds(i, 16)][0]
            pltpu.sync_copy(pend_sh.at[pl.ds(p * D, D)], rowbuf)
            csp = jnp.full((16,), col, jnp.int32)
            for q in range(4):
                x = rowbuf[pl.ds(q * 16, 16)]
                if tail:
                    in_a = csp < TAIL_A
                    plsc.store_scatter(tba, [q * 16 + lanes, csp], x,
                                       mask=in_a)
                    plsc.store_scatter(tbb, [q * 16 + lanes, csp - TAIL_A],
                                       x, mask=~in_a)
                else:
                    plsc.store_scatter(bufs[b], [q * 16 + lanes, csp], x)
            return 0

        lax.fori_loop(0, cw_, apply_one, 0)

    fire_in(w, 0, False)
    fire_in(w + NW, 1, False)

    for k in range(KMAX):
        b = k % 2
        c = w + NW * k
        is_real = c < NCHK
        is_tail = c == NCHK - 1
        cn = c + 2 * NW

        @pl.when(jnp.logical_and(is_real, jnp.logical_not(is_tail)))
        def _():
            wait_in(b, False)
            patch_chunk(k, b)
            fire_out(c, b, False)

            @pl.when(cn < NCHK - 1)
            def _():
                wait_out(b, False)
                fire_in(cn, b, False)

            @pl.when(cn == NCHK - 1)
            def _():
                wait_out(b, False)
                fire_in(cn, b, True)

        @pl.when(is_tail)
        def _():
            wait_in(b, True)
            patch_chunk(k, b, tail=True)
            fire_out(c, b, True)

    for k in range(KMAX):
        b = k % 2
        c = w + NW * k
        pending = jnp.logical_and(c < NCHK, c + 2 * NW >= NCHK)

        @pl.when(jnp.logical_and(pending, c != NCHK - 1))
        def _():
            wait_out(b, False)

        @pl.when(jnp.logical_and(pending, c == NCHK - 1))
        def _():
            wait_out(b, True)


def kernel(bank, idx, val):
    mesh = plsc.VectorSubcoreMesh(core_axis_name="c", subcore_axis_name="s")
    fused = _mpmd._mpmd_map(
        [(mesh, _body)],
        out_types=jax.ShapeDtypeStruct((D, N), jnp.float32),
        scratch_types=[
            pltpu.VMEM_SHARED((B * D,), jnp.float32),
            pltpu.VMEM((B,), jnp.int32),
            pltpu.VMEM((B + 16,), jnp.int32),
            pltpu.VMEM((B + 16,), jnp.int32),
            pltpu.VMEM((SLOTS + 16,), jnp.int32),
            pltpu.VMEM((D, 128), jnp.float32),
            pltpu.VMEM((128 * D,), jnp.float32),
            pltpu.VMEM((CW + 16,), jnp.int32),
            pltpu.VMEM((CW + 16,), jnp.int32),
            pltpu.VMEM((D,), jnp.float32),
            pltpu.VMEM((D, CW), jnp.float32),
            pltpu.VMEM((D, CW), jnp.float32),
            pltpu.VMEM((D, TAIL_A), jnp.float32),
            pltpu.VMEM((D, TAIL_B), jnp.float32),
            pltpu.SemaphoreType.DMA,
            pltpu.SemaphoreType.DMA,
            pltpu.SemaphoreType.DMA,
            pltpu.SemaphoreType.DMA,
        ],
        compiler_params=pltpu.CompilerParams(needs_layout_passes=False),
        name="sc_bank_scatter_fused",
    )
    outT = fused(bank.T, idx, val.T)
    return outT.T


# speedup vs baseline: 1.6825x; 1.0058x over previous
import jax
import jax.numpy as jnp
from jax import lax
from jax.experimental import pallas as pl
from jax.experimental.pallas import tpu as pltpu
from jax.experimental.pallas import tpu_sc as plsc
from jax._src.pallas import mpmd as _mpmd

N = 100000
D = 64
B = 4096
NC = 2
NS = 16
NW = NC * NS
NGV = B // 16
CW = 512
NCHK = (N + CW - 1) // CW
KMAX = (NCHK + NW - 1) // NW
TAIL_A = 128
TAIL_B = N - (NCHK - 1) * CW - TAIL_A
PPS = B // NS
SLOTS = KMAX * CW


def _shift_dup_mask(v, lanes):
    later = jnp.zeros((16,), jnp.bool_)
    for k in range(1, 16):
        ids = jnp.minimum(lanes + k, 15)
        sh = jnp.take_along_axis(v, ids, axis=0)
        later = later | ((v == sh) & ((lanes + k) < 16))
    return later


def _body(bankT_hbm, idx_hbm, valT_hbm, outT_hbm,
          pend_sh, idx_v, sel_idx, sel_pos, stamp,
          vblk, flatbuf, wcol, wpos, rowbuf,
          cb0, cb1, tba, tbb, si0, si1, so0, so1):
    s = lax.axis_index("s")
    w = s * NC + lax.axis_index("c")
    lanes = lax.iota(jnp.int32, 16)

    for h in range(2):
        p0 = s * PPS + h * 128
        pltpu.sync_copy(valT_hbm.at[:, pl.ds(p0, 128)], vblk)

        for half in range(2):
            def xpose(c, _):
                csp = jnp.full((16,), half * 64 + c, jnp.int32)
                for q in range(4):
                    x = plsc.load_gather(vblk, [q * 16 + lanes, csp])
                    flatbuf[pl.ds(c * D + q * 16, 16)] = x
                return 0

            lax.fori_loop(0, 64, xpose, 0)
            pltpu.sync_copy(flatbuf,
                            pend_sh.at[pl.ds((p0 + half * 64) * D, 64 * D)])

    pltpu.sync_copy(idx_hbm, idx_v)

    def initstamp(t, _):
        stamp[pl.ds(t * 16, 16)] = jnp.full((16,), -1, jnp.int32)
        return 0

    lax.fori_loop(0, SLOTS // 16, initstamp, 0)

    def scan_body(k, cnt):
        v = idx_v[pl.ds(k * 16, 16)]
        m = ((v >> 9) & 31) == w
        pos = k * 16 + lanes
        ones = jnp.where(m, 1, 0)
        dst = cnt + plsc.cumsum(ones) - 1
        dst = jnp.where(m, dst, 0)
        plsc.store_scatter(sel_idx, [dst], v, mask=m)
        plsc.store_scatter(sel_pos, [dst], pos, mask=m)
        return cnt + jnp.sum(ones)

    cnt = lax.fori_loop(0, NGV, scan_body, jnp.int32(0), unroll=4)
    nsel = (cnt + 15) // 16

    def stamp_body(g, _):
        off = g * 16
        v = sel_idx[pl.ds(off, 16)]
        p = sel_pos[pl.ds(off, 16)]
        valid = (off + lanes) < cnt
        vc = jnp.where(valid, v, -1)
        keep = valid & ~_shift_dup_mask(vc, lanes)
        slot = (vc >> 14) * CW + (vc & (CW - 1))
        slot = jnp.where(keep, slot, 0)
        plsc.store_scatter(stamp, [slot], p, mask=keep)
        return 0

    lax.fori_loop(0, nsel, stamp_body, 0)
    plsc.subcore_barrier()

    bufs = [cb0, cb1]
    sins = [si0, si1]
    souts = [so0, so1]

    def fire_in(c, b, tail):
        if tail:
            pltpu.async_copy(bankT_hbm.at[:, pl.ds((NCHK - 1) * CW, TAIL_A)],
                             tba, sins[b])
            pltpu.async_copy(
                bankT_hbm.at[:, pl.ds((NCHK - 1) * CW + TAIL_A, TAIL_B)],
                tbb, sins[b])
        else:
            pltpu.async_copy(bankT_hbm.at[:, pl.ds(c * CW, CW)], bufs[b],
                             sins[b])

    def wait_in(b, tail):
        if tail:
            pltpu.make_async_copy(bankT_hbm.at[:, pl.ds(0, TAIL_A)],
                                  tba, sins[b]).wait()
            pltpu.make_async_copy(
                bankT_hbm.at[:, pl.ds((NCHK - 1) * CW + TAIL_A, TAIL_B)],
                tbb, sins[b]).wait()
        else:
            pltpu.make_async_copy(bankT_hbm.at[:, pl.ds(0, CW)], bufs[b],
                                  sins[b]).wait()

    def fire_out(c, b, tail):
        if tail:
            pltpu.async_copy(tba,
                             outT_hbm.at[:, pl.ds((NCHK - 1) * CW, TAIL_A)],
                             souts[b])
            pltpu.async_copy(
                tbb,
                outT_hbm.at[:, pl.ds((NCHK - 1) * CW + TAIL_A, TAIL_B)],
                souts[b])
        else:
            pltpu.async_copy(bufs[b], outT_hbm.at[:, pl.ds(c * CW, CW)],
                             souts[b])

    def wait_out(b, tail):
        if tail:
            pltpu.make_async_copy(tba, outT_hbm.at[:, pl.ds(0, TAIL_A)],
                                  souts[b]).wait()
            pltpu.make_async_copy(
                tbb,
                outT_hbm.at[:, pl.ds((NCHK - 1) * CW + TAIL_A, TAIL_B)],
                souts[b]).wait()
        else:
            pltpu.make_async_copy(bufs[b], outT_hbm.at[:, pl.ds(0, CW)],
                                  souts[b]).wait()

    def patch_chunk(k, b, tail=False):
        def walk(t, cw_):
            sl = stamp[pl.ds(k * CW + t * 16, 16)]
            m = sl >= 0
            ones = jnp.where(m, 1, 0)
            dst = cw_ + plsc.cumsum(ones) - 1
            dst = jnp.where(m, dst, 0)
            plsc.store_scatter(wcol, [dst], t * 16 + lanes, mask=m)
            plsc.store_scatter(wpos, [dst], sl, mask=m)
            return cw_ + jnp.sum(ones)

        cw_ = lax.fori_loop(0, CW // 16, walk, jnp.int32(0))

        def apply_one(i, _):
            col = wcol[pl.ds(i, 16)][0]
            p = wpos[pl.ds(i, 16)][0]
            pltpu.sync_copy(pend_sh.at[pl.ds(p * D, D)], rowbuf)
            csp = jnp.full((16,), col, jnp.int32)
            for q in range(4):
                x = rowbuf[pl.ds(q * 16, 16)]
                if tail:
                    in_a = csp < TAIL_A
                    plsc.store_scatter(tba, [q * 16 + lanes, csp], x,
                                       mask=in_a)
                    plsc.store_scatter(tbb, [q * 16 + lanes, csp - TAIL_A],
                                       x, mask=~in_a)
                else:
                    plsc.store_scatter(bufs[b], [q * 16 + lanes, csp], x)
            return 0

        lax.fori_loop(0, cw_, apply_one, 0)

    fire_in(w, 0, False)
    fire_in(w + NW, 1, False)

    for k in range(KMAX):
        b = k % 2
        c = w + NW * k
        is_real = c < NCHK
        is_tail = c == NCHK - 1
        cn = c + 2 * NW

        @pl.when(jnp.logical_and(is_real, jnp.logical_not(is_tail)))
        def _():
            wait_in(b, False)
            patch_chunk(k, b)
            fire_out(c, b, False)

            @pl.when(cn < NCHK - 1)
            def _():
                wait_out(b, False)
                fire_in(cn, b, False)

            @pl.when(cn == NCHK - 1)
            def _():
                wait_out(b, False)
                fire_in(cn, b, True)

        @pl.when(is_tail)
        def _():
            wait_in(b, True)
            patch_chunk(k, b, tail=True)
            fire_out(c, b, True)

    for k in range(KMAX):
        b = k % 2
        c = w + NW * k
        pending = jnp.logical_and(c < NCHK, c + 2 * NW >= NCHK)

        @pl.when(jnp.logical_and(pending, c != NCHK - 1))
        def _():
            wait_out(b, False)

        @pl.when(jnp.logical_and(pending, c == NCHK - 1))
        def _():
            wait_out(b, True)


def kernel(bank, idx, val):
    mesh = plsc.VectorSubcoreMesh(core_axis_name="c", subcore_axis_name="s")
    fused = _mpmd._mpmd_map(
        [(mesh, _body)],
        out_types=jax.ShapeDtypeStruct((D, N), jnp.float32),
        scratch_types=[
            pltpu.VMEM_SHARED((B * D,), jnp.float32),
            pltpu.VMEM((B,), jnp.int32),
            pltpu.VMEM((B + 16,), jnp.int32),
            pltpu.VMEM((B + 16,), jnp.int32),
            pltpu.VMEM((SLOTS + 16,), jnp.int32),
            pltpu.VMEM((D, 128), jnp.float32),
            pltpu.VMEM((64 * D,), jnp.float32),
            pltpu.VMEM((CW + 16,), jnp.int32),
            pltpu.VMEM((CW + 16,), jnp.int32),
            pltpu.VMEM((D,), jnp.float32),
            pltpu.VMEM((D, CW), jnp.float32),
            pltpu.VMEM((D, CW), jnp.float32),
            pltpu.VMEM((D, TAIL_A), jnp.float32),
            pltpu.VMEM((D, TAIL_B), jnp.float32),
            pltpu.SemaphoreType.DMA,
            pltpu.SemaphoreType.DMA,
            pltpu.SemaphoreType.DMA,
            pltpu.SemaphoreType.DMA,
        ],
        compiler_params=pltpu.CompilerParams(needs_layout_passes=False),
        name="sc_bank_scatter_fused",
    )
    outT = fused(bank.T, idx, val.T)
    return outT.T
